# baseline (speedup 1.0000x reference)
import functools

import jax
import jax.numpy as jnp
import numpy as np
from jax.experimental import pallas as pl

B = 32
N = 512
DIN = 64
H = 64
TOPK = 32

_INT_MIN = np.int32(-2147483648)

_RADIX_PASSES = 26

_BPP = 4


def _body(x_ref, st_ref, wlin_ref, blin_ref, wq_ref, wo_ref, lng_ref,
          lnb_ref, wg1_ref, bg1_ref, wu_ref, bu_ref, h_ref, aout_ref):
    f32 = jnp.float32
    dot = functools.partial(jnp.dot, preferred_element_type=f32)

    ias_l, st_l, v_l, s_l, k32_l = [], [], [], [], []
    for j in range(_BPP):
        xbT = x_ref[j].T
        stT = st_ref[j].T
        xhT = dot(wlin_ref[...], xbT) + blin_ref[...]
        iasT = jnp.concatenate([xhT, stT], axis=0)
        qkvT = dot(wq_ref[...], iasT)
        qT = qkvT[:H]
        kT = qkvT[H:2 * H]
        vT = qkvT[2 * H:]
        sT = dot(kT.T, qT) * (1.0 / 8.0)
        i = jax.lax.bitcast_convert_type(sT, jnp.int32)
        k32 = jnp.where(i < 0, _INT_MIN - i, i)
        ias_l.append(iasT); st_l.append(stT); v_l.append(vT)
        s_l.append(sT); k32_l.append(k32)

    kf = float(TOPK)

    ones_row = jnp.ones((1, N), f32)

    def count_ge(j, cand_s):
        ge = k32_l[j] >= cand_s
        return jnp.dot(ones_row, ge.astype(f32), preferred_element_type=f32)

    ts_l = [jnp.where(count_ge(j, jnp.int32(0)) >= kf, jnp.int32(0), _INT_MIN)
            for j in range(_BPP)]
    for b in range(30, 31 - _RADIX_PASSES, -1):
        bit = np.int32(1 << b)
        for j in range(_BPP):
            cand_s = ts_l[j] | bit
            ts_l[j] = jnp.where(count_ge(j, cand_s) >= kf, cand_s, ts_l[j])

    for j in range(_BPP):
        sT, k32, iasT, stT, vT = s_l[j], k32_l[j], ias_l[j], st_l[j], v_l[j]
        mask = k32 >= ts_l[j]
        smax = jnp.max(sT, axis=0, keepdims=True)
        e = jnp.where(mask, jnp.exp(sT - smax), 0.0)
        denom = jnp.sum(e, axis=0, keepdims=True)
        attnT = e / denom

        ctxT = dot(vT, attnT)
        preT = iasT + dot(wo_ref[...], ctxT)
        mu = jnp.mean(preT, axis=0, keepdims=True)
        dev = preT - mu
        var = jnp.mean(dev * dev, axis=0, keepdims=True)
        xxT = dev * jax.lax.rsqrt(var + 1e-6) * lng_ref[...] + lnb_ref[...]

        def gcn(xfT, w_ref, b_ref):
            xwT = dot(w_ref[...], xfT)
            return 0.5 * (dot(xwT, attnT) + xwT) + b_ref[...]

        zr = jax.nn.sigmoid(gcn(xxT, wg1_ref, bg1_ref))
        z = zr[:H]
        r = zr[H:]
        candT = jnp.concatenate([xxT, z * stT], axis=0)
        hcT = jnp.tanh(gcn(candT, wu_ref, bu_ref))
        h_ref[j] = (r * stT + (1.0 - r) * hcT).T
        aout_ref[j] = attnT


def kernel(x, state, W_lin, b_lin, Wq, Wk, Wv, Wo, ln_g, ln_b,
           Wg1, bg1, Wg2, bg2, Wu, bu):
    col = lambda a: a.reshape(-1, 1)
    full = lambda shp: pl.BlockSpec(shp, lambda b: (0,) * len(shp))
    per_b = lambda shp: pl.BlockSpec((_BPP,) + shp, lambda b: (b, 0, 0))

    wqkvT = jnp.concatenate([Wq.T, Wk.T, Wv.T], axis=0)
    wg12T = jnp.concatenate([Wg1.T, Wg2.T], axis=0)
    bg12 = jnp.concatenate([bg1, bg2], axis=0)

    h, a_out = pl.pallas_call(
        _body,
        grid=(B // _BPP,),
        in_specs=[
            per_b((N, DIN)),
            per_b((N, H)),
            full((H, DIN)),
            full((H, 1)),
            full((3 * H, 2 * H)),
            full((2 * H, H)),
            full((2 * H, 1)),
            full((2 * H, 1)),
            full((2 * H, 2 * H)),
            full((2 * H, 1)),
            full((H, 3 * H)),
            full((H, 1)),
        ],
        out_specs=[
            per_b((N, H)),
            per_b((N, N)),
        ],
        out_shape=[
            jax.ShapeDtypeStruct((B, N, H), jnp.float32),
            jax.ShapeDtypeStruct((B, N, N), jnp.float32),
        ],
    )(x, state, W_lin.T, col(b_lin), wqkvT, Wo.T, col(ln_g), col(ln_b),
      wg12T, col(bg12), Wu.T, col(bu))
    return h, a_out


# speedup vs baseline: 1.0114x; 1.0114x over previous
import functools

import jax
import jax.numpy as jnp
import numpy as np
from jax.experimental import pallas as pl

B = 32
N = 512
DIN = 64
H = 64
TOPK = 32

_INT_MIN = np.int32(-2147483648)

_RADIX_PASSES = 26

_BPP = 4


def _body(x_ref, st_ref, wlin_ref, blin_ref, wq_ref, wk_ref, wv_ref,
          wo_ref, lng_ref, lnb_ref, wg1_ref, bg1_ref, wg2_ref, bg2_ref,
          wu_ref, bu_ref, h_ref, aout_ref):
    f32 = jnp.float32
    dot = functools.partial(jnp.dot, preferred_element_type=f32)

    ias_l, st_l, v_l, s_l, k32_l = [], [], [], [], []
    for j in range(_BPP):
        xbT = x_ref[j].T
        stT = st_ref[j].T
        xhT = dot(wlin_ref[...], xbT) + blin_ref[...]
        iasT = jnp.concatenate([xhT, stT], axis=0)
        qT = dot(wq_ref[...], iasT)
        kT = dot(wk_ref[...], iasT)
        vT = dot(wv_ref[...], iasT)
        sT = dot(kT.T, qT) * (1.0 / 8.0)
        i = jax.lax.bitcast_convert_type(sT, jnp.int32)
        k32 = jnp.where(i < 0, _INT_MIN - i, i)
        ias_l.append(iasT); st_l.append(stT); v_l.append(vT)
        s_l.append(sT); k32_l.append(k32)

    kf = float(TOPK)

    ones_row = jnp.ones((1, N), f32)

    def count_ge(j, cand_s):
        ge = k32_l[j] >= cand_s
        return jnp.dot(ones_row, ge.astype(f32), preferred_element_type=f32)

    ts_l = [jnp.where(count_ge(j, jnp.int32(0)) >= kf, jnp.int32(0), _INT_MIN)
            for j in range(_BPP)]
    for b in range(30, 31 - _RADIX_PASSES, -1):
        bit = np.int32(1 << b)
        for j in range(_BPP):
            cand_s = ts_l[j] | bit
            ts_l[j] = jnp.where(count_ge(j, cand_s) >= kf, cand_s, ts_l[j])

    for j in range(_BPP):
        sT, k32, iasT, stT, vT = s_l[j], k32_l[j], ias_l[j], st_l[j], v_l[j]
        mask = k32 >= ts_l[j]
        smax = jnp.max(sT, axis=0, keepdims=True)
        e = jnp.where(mask, jnp.exp(sT - smax), 0.0)
        denom = jnp.sum(e, axis=0, keepdims=True)
        attnT = e / denom

        ctxT = dot(vT, attnT)
        preT = iasT + dot(wo_ref[...], ctxT)
        mu = jnp.mean(preT, axis=0, keepdims=True)
        dev = preT - mu
        var = jnp.mean(dev * dev, axis=0, keepdims=True)
        xxT = dev * jax.lax.rsqrt(var + 1e-6) * lng_ref[...] + lnb_ref[...]

        def gcn(xfT, w_ref, b_ref):
            xwT = dot(w_ref[...], xfT)
            return 0.5 * (dot(xwT, attnT) + xwT) + b_ref[...]

        z = jax.nn.sigmoid(gcn(xxT, wg1_ref, bg1_ref))
        r = jax.nn.sigmoid(gcn(xxT, wg2_ref, bg2_ref))
        candT = jnp.concatenate([xxT, z * stT], axis=0)
        hcT = jnp.tanh(gcn(candT, wu_ref, bu_ref))
        h_ref[j] = (r * stT + (1.0 - r) * hcT).T
        aout_ref[j] = attnT


def kernel(x, state, W_lin, b_lin, Wq, Wk, Wv, Wo, ln_g, ln_b,
           Wg1, bg1, Wg2, bg2, Wu, bu):
    col = lambda a: a.reshape(-1, 1)
    full = lambda shp: pl.BlockSpec(shp, lambda b: (0,) * len(shp))
    per_b = lambda shp: pl.BlockSpec((_BPP,) + shp, lambda b: (b, 0, 0))

    h, a_out = pl.pallas_call(
        _body,
        grid=(B // _BPP,),
        in_specs=[
            per_b((N, DIN)),
            per_b((N, H)),
            full((H, DIN)),
            full((H, 1)),
            full((H, 2 * H)),
            full((H, 2 * H)),
            full((H, 2 * H)),
            full((2 * H, H)),
            full((2 * H, 1)),
            full((2 * H, 1)),
            full((H, 2 * H)),
            full((H, 1)),
            full((H, 2 * H)),
            full((H, 1)),
            full((H, 3 * H)),
            full((H, 1)),
        ],
        out_specs=[
            per_b((N, H)),
            per_b((N, N)),
        ],
        out_shape=[
            jax.ShapeDtypeStruct((B, N, H), jnp.float32),
            jax.ShapeDtypeStruct((B, N, N), jnp.float32),
        ],
    )(x, state, W_lin.T, col(b_lin), Wq.T, Wk.T, Wv.T, Wo.T, col(ln_g),
      col(ln_b), Wg1.T, col(bg1), Wg2.T, col(bg2), Wu.T, col(bu))
    return h, a_out
